# baseline (speedup 1.0000x reference)
import functools

import jax
import jax.numpy as jnp
from jax import lax
from jax.experimental import pallas as pl
from jax.experimental.pallas import tpu as pltpu
from jax.experimental.pallas import tpu_sc as plsc

EMB_D = 8


def _make_sc_gather(B, n_workers, num_cores):
  b_per_w = B // n_workers
  mesh = plsc.VectorSubcoreMesh(core_axis_name="c", subcore_axis_name="s")

  @functools.partial(
      pl.kernel,
      mesh=mesh,
      compiler_params=pltpu.CompilerParams(use_tc_tiling_on_sc=False,
                                           needs_layout_passes=False,
                                           disable_semaphore_checks=True),
      out_type=jax.ShapeDtypeStruct((2 * EMB_D * B,), jnp.float32),
      scratch_types=[
          pltpu.VMEM((b_per_w,), jnp.int32),
          pltpu.VMEM((b_per_w,), jnp.int32),
          pltpu.VMEM((16, EMB_D, 128), jnp.float32),
          pltpu.VMEM((16, EMB_D, 128), jnp.float32),
          pltpu.VMEM((2 * EMB_D * b_per_w,), jnp.float32),
          pltpu.SemaphoreType.DMA,
          pltpu.SemaphoreType.DMA,
      ],
  )
  def sc_gather(ut_hbm, uid_hbm, tt_hbm, tid_hbm, out, uidx_v, tidx_v,
                tiles_u, tiles_t, cols_v, sem_u, sem_t):
    wid = lax.axis_index("s") * num_cores + lax.axis_index("c")
    base = wid * b_per_w
    pltpu.sync_copy(uid_hbm.at[pl.ds(base, b_per_w)], uidx_v)
    pltpu.sync_copy(tid_hbm.at[pl.ds(base, b_per_w)], tidx_v)

    lanej = lax.iota(jnp.int32, 16)

    def chunk(t, _):
      e0 = t * 16
      uv = uidx_v[pl.ds(e0, 16)]
      tv = tidx_v[pl.ds(e0, 16)]
      cps = []
      for j in range(16):
        tu = lax.shift_right_logical(uv[j], 7) * 128
        tt = lax.shift_right_logical(tv[j], 7) * 128
        cps.append(pltpu.async_copy(ut_hbm.at[:, pl.ds(tu, 128)],
                                    tiles_u.at[j], sem_u))
        cps.append(pltpu.async_copy(tt_hbm.at[:, pl.ds(tt, 128)],
                                    tiles_t.at[j], sem_t))
      for cp in cps:
        cp.wait()
      cu = lax.bitwise_and(uv, 127)
      ct = lax.bitwise_and(tv, 127)
      for k in range(EMB_D):
        ksplat = jnp.full((16,), k, jnp.int32)
        valu = plsc.load_gather(tiles_u, [lanej, ksplat, cu])
        valt = plsc.load_gather(tiles_t, [lanej, ksplat, ct])
        cols_v[pl.ds(k * b_per_w + e0, 16)] = valu
        cols_v[pl.ds((EMB_D + k) * b_per_w + e0, 16)] = valt
      return _

    lax.fori_loop(0, b_per_w // 16, chunk, None)

    for k in range(2 * EMB_D):
      pltpu.sync_copy(cols_v.at[pl.ds(k * b_per_w, b_per_w)],
                      out.at[pl.ds(k * B + base, b_per_w)])

  return sc_gather


def _mlp_body(emb_ref, uc_ref, ac_ref,
              w1_ref, b1_ref, w2_ref, b2_ref, w3_ref, b3_ref, w4_ref, b4_ref,
              out_ref):
  def dg(w, x):
    return lax.dot_general(w, x, (((0,), (0,)), ((), ())),
                           precision=lax.Precision.HIGHEST,
                           preferred_element_type=jnp.float32)

  h = dg(w1_ref[0:EMB_D, :], emb_ref[0:EMB_D, :])
  h += dg(w1_ref[EMB_D:2 * EMB_D, :], emb_ref[EMB_D:2 * EMB_D, :])
  h += dg(w1_ref[2 * EMB_D:2 * EMB_D + 64, :], uc_ref[...])
  h += dg(w1_ref[2 * EMB_D + 64:, :], ac_ref[...])
  h = jnp.maximum(h + b1_ref[...][:, None], 0.0)
  h = jnp.maximum(dg(w2_ref[...], h) + b2_ref[...][:, None], 0.0)
  h = jnp.maximum(dg(w3_ref[...], h) + b3_ref[...][:, None], 0.0)
  out = dg(w4_ref[...], h) + b4_ref[...][:, None]
  out_ref[...] = out[0, :]


def _mlp_call(embT, ucT, acT, W1, b1, W2, b2, W3, b3, W4, b4, blk):
  B = embT.shape[1]
  grid = (B // blk,)

  def col_spec(d):
    return pl.BlockSpec((d, blk), lambda i: (0, i))

  def full_spec(shape):
    nd = len(shape)
    return pl.BlockSpec(shape, lambda i: (0,) * nd)

  return pl.pallas_call(
      _mlp_body,
      grid=grid,
      in_specs=[
          col_spec(2 * EMB_D), col_spec(64), col_spec(64),
          full_spec(W1.shape), full_spec(b1.shape),
          full_spec(W2.shape), full_spec(b2.shape),
          full_spec(W3.shape), full_spec(b3.shape),
          full_spec(W4.shape), full_spec(b4.shape),
      ],
      out_specs=pl.BlockSpec((blk,), lambda i: (i,)),
      out_shape=jax.ShapeDtypeStruct((B,), jnp.float32),
  )(embT, ucT, acT, W1, b1, W2, b2, W3, b3, W4, b4)


@jax.jit
def kernel(user_id, artist_id, user_country, artist_country, U, T,
           W1, b1, W2, b2, W3, b3, W4, b4):
  B = user_id.shape[0]
  mesh = plsc.VectorSubcoreMesh(core_axis_name="c", subcore_axis_name="s")
  n_workers = mesh.num_cores * mesh.num_subcores
  gather = _make_sc_gather(B, n_workers, mesh.num_cores)
  embT = gather(U.T, user_id, T.T, artist_id).reshape(2 * EMB_D, B)
  return _mlp_call(embT, user_country.T, artist_country.T,
                   W1, b1, W2, b2, W3, b3, W4, b4, blk=2048)


# speedup vs baseline: 1.6155x; 1.6155x over previous
import functools

import jax
import jax.numpy as jnp
from jax import lax
from jax.experimental import pallas as pl
from jax.experimental.pallas import tpu as pltpu
from jax.experimental.pallas import tpu_sc as plsc

EMB_D = 8


def _make_sc_gather(B, n_workers, num_cores):
  b_per_w = B // n_workers
  mesh = plsc.VectorSubcoreMesh(core_axis_name="c", subcore_axis_name="s")

  @functools.partial(
      pl.kernel,
      mesh=mesh,
      compiler_params=pltpu.CompilerParams(use_tc_tiling_on_sc=False,
                                           needs_layout_passes=False,
                                           disable_semaphore_checks=True),
      out_type=jax.ShapeDtypeStruct((2 * EMB_D * B,), jnp.float32),
      scratch_types=[
          pltpu.VMEM((b_per_w,), jnp.int32),
          pltpu.VMEM((b_per_w,), jnp.int32),
          pltpu.VMEM((b_per_w, EMB_D), jnp.float32),
          pltpu.VMEM((b_per_w, EMB_D), jnp.float32),
          pltpu.VMEM((2 * EMB_D * b_per_w,), jnp.float32),
          pltpu.SemaphoreType.DMA,
          pltpu.SemaphoreType.DMA,
      ],
  )
  def sc_gather(u_hbm, uid_hbm, t_hbm, tid_hbm, out, uidx_v, tidx_v,
                urows_v, trows_v, cols_v, sem_u, sem_t):
    wid = lax.axis_index("s") * num_cores + lax.axis_index("c")
    base = wid * b_per_w
    pltpu.sync_copy(uid_hbm.at[pl.ds(base, b_per_w)], uidx_v)
    pltpu.sync_copy(tid_hbm.at[pl.ds(base, b_per_w)], tidx_v)

    cp_u = pltpu.async_copy(u_hbm.at[uidx_v], urows_v, sem_u)
    cp_t = pltpu.async_copy(t_hbm.at[tidx_v], trows_v, sem_t)
    cp_u.wait()
    cp_t.wait()

    lanej = lax.iota(jnp.int32, 16)

    def chunk(t, _):
      e0 = t * 16
      jv = lanej + e0
      for k in range(EMB_D):
        ksplat = jnp.full((16,), k, jnp.int32)
        valu = plsc.load_gather(urows_v, [jv, ksplat])
        valt = plsc.load_gather(trows_v, [jv, ksplat])
        cols_v[pl.ds(k * b_per_w + e0, 16)] = valu
        cols_v[pl.ds((EMB_D + k) * b_per_w + e0, 16)] = valt
      return _

    lax.fori_loop(0, b_per_w // 16, chunk, None)

    for k in range(2 * EMB_D):
      pltpu.sync_copy(cols_v.at[pl.ds(k * b_per_w, b_per_w)],
                      out.at[pl.ds(k * B + base, b_per_w)])

  return sc_gather


def _mlp_body(emb_ref, uc_ref, ac_ref,
              w1_ref, b1_ref, w2_ref, b2_ref, w3_ref, b3_ref, w4_ref, b4_ref,
              out_ref):
  def dg(w, x):
    return lax.dot_general(w, x, (((0,), (0,)), ((), ())),
                           precision=lax.Precision.HIGHEST,
                           preferred_element_type=jnp.float32)

  h = dg(w1_ref[0:EMB_D, :], emb_ref[0:EMB_D, :])
  h += dg(w1_ref[EMB_D:2 * EMB_D, :], emb_ref[EMB_D:2 * EMB_D, :])
  h += dg(w1_ref[2 * EMB_D:2 * EMB_D + 64, :], uc_ref[...])
  h += dg(w1_ref[2 * EMB_D + 64:, :], ac_ref[...])
  h = jnp.maximum(h + b1_ref[...][:, None], 0.0)
  h = jnp.maximum(dg(w2_ref[...], h) + b2_ref[...][:, None], 0.0)
  h = jnp.maximum(dg(w3_ref[...], h) + b3_ref[...][:, None], 0.0)
  out = dg(w4_ref[...], h) + b4_ref[...][:, None]
  out_ref[...] = out[0, :]


def _mlp_call(embT, ucT, acT, W1, b1, W2, b2, W3, b3, W4, b4, blk):
  B = embT.shape[1]
  grid = (B // blk,)

  def col_spec(d):
    return pl.BlockSpec((d, blk), lambda i: (0, i))

  def full_spec(shape):
    nd = len(shape)
    return pl.BlockSpec(shape, lambda i: (0,) * nd)

  return pl.pallas_call(
      _mlp_body,
      grid=grid,
      in_specs=[
          col_spec(2 * EMB_D), col_spec(64), col_spec(64),
          full_spec(W1.shape), full_spec(b1.shape),
          full_spec(W2.shape), full_spec(b2.shape),
          full_spec(W3.shape), full_spec(b3.shape),
          full_spec(W4.shape), full_spec(b4.shape),
      ],
      out_specs=pl.BlockSpec((blk,), lambda i: (i,)),
      out_shape=jax.ShapeDtypeStruct((B,), jnp.float32),
  )(embT, ucT, acT, W1, b1, W2, b2, W3, b3, W4, b4)


@jax.jit
def kernel(user_id, artist_id, user_country, artist_country, U, T,
           W1, b1, W2, b2, W3, b3, W4, b4):
  B = user_id.shape[0]
  mesh = plsc.VectorSubcoreMesh(core_axis_name="c", subcore_axis_name="s")
  n_workers = mesh.num_cores * mesh.num_subcores
  gather = _make_sc_gather(B, n_workers, mesh.num_cores)
  embT = gather(U, user_id, T, artist_id).reshape(2 * EMB_D, B)
  return _mlp_call(embT, user_country.T, artist_country.T,
                   W1, b1, W2, b2, W3, b3, W4, b4, blk=2048)


# speedup vs baseline: 12.6139x; 7.8082x over previous
import functools

import jax
import jax.numpy as jnp
from jax import lax
from jax.experimental import pallas as pl
from jax.experimental.pallas import tpu as pltpu
from jax.experimental.pallas import tpu_sc as plsc

EMB_D = 8


def _make_sc_gather(B, n_workers, num_cores):
  b_per_w = B // n_workers
  mesh = plsc.VectorSubcoreMesh(core_axis_name="c", subcore_axis_name="s")

  @functools.partial(
      pl.kernel,
      mesh=mesh,
      compiler_params=pltpu.CompilerParams(use_tc_tiling_on_sc=True,
                                           needs_layout_passes=False,
                                           disable_semaphore_checks=True),
      out_type=jax.ShapeDtypeStruct((2 * EMB_D * B,), jnp.float32),
      scratch_types=[
          pltpu.VMEM((b_per_w,), jnp.int32),
          pltpu.VMEM((b_per_w,), jnp.int32),
          pltpu.VMEM((16, EMB_D, 128), jnp.float32),
          pltpu.VMEM((16, EMB_D, 128), jnp.float32),
          pltpu.VMEM((2 * EMB_D * b_per_w,), jnp.float32),
          pltpu.SemaphoreType.DMA,
          pltpu.SemaphoreType.DMA,
      ],
  )
  def sc_gather(ut_hbm, uid_hbm, tt_hbm, tid_hbm, out, uidx_v, tidx_v,
                tiles_u, tiles_t, cols_v, sem_u, sem_t):
    wid = lax.axis_index("s") * num_cores + lax.axis_index("c")
    base = wid * b_per_w
    pltpu.sync_copy(uid_hbm.at[pl.ds(base, b_per_w)], uidx_v)
    pltpu.sync_copy(tid_hbm.at[pl.ds(base, b_per_w)], tidx_v)

    lanej = lax.iota(jnp.int32, 16)

    def chunk(t, _):
      e0 = t * 16
      uv = uidx_v[pl.ds(e0, 16)]
      tv = tidx_v[pl.ds(e0, 16)]
      cps = []
      for j in range(16):
        tu = lax.shift_right_logical(uv[j], 7) * 128
        tt = lax.shift_right_logical(tv[j], 7) * 128
        cps.append(pltpu.async_copy(ut_hbm.at[:, pl.ds(tu, 128)],
                                    tiles_u.at[j], sem_u))
        cps.append(pltpu.async_copy(tt_hbm.at[:, pl.ds(tt, 128)],
                                    tiles_t.at[j], sem_t))
      for cp in cps:
        cp.wait()
      cu = lax.bitwise_and(uv, 127)
      ct = lax.bitwise_and(tv, 127)
      for k in range(EMB_D):
        ksplat = jnp.full((16,), k, jnp.int32)
        valu = plsc.load_gather(tiles_u, [lanej, ksplat, cu])
        valt = plsc.load_gather(tiles_t, [lanej, ksplat, ct])
        cols_v[pl.ds(k * b_per_w + e0, 16)] = valu
        cols_v[pl.ds((EMB_D + k) * b_per_w + e0, 16)] = valt
      return _

    lax.fori_loop(0, b_per_w // 16, chunk, None)

    for k in range(2 * EMB_D):
      pltpu.sync_copy(cols_v.at[pl.ds(k * b_per_w, b_per_w)],
                      out.at[pl.ds(k * B + base, b_per_w)])

  return sc_gather


def _mlp_body(emb_ref, uc_ref, ac_ref,
              w1_ref, b1_ref, w2_ref, b2_ref, w3_ref, b3_ref, w4_ref, b4_ref,
              out_ref):
  def dg(w, x):
    return lax.dot_general(w, x, (((0,), (0,)), ((), ())),
                           precision=lax.Precision.HIGHEST,
                           preferred_element_type=jnp.float32)

  h = dg(w1_ref[0:EMB_D, :], emb_ref[0:EMB_D, :])
  h += dg(w1_ref[EMB_D:2 * EMB_D, :], emb_ref[EMB_D:2 * EMB_D, :])
  h += dg(w1_ref[2 * EMB_D:2 * EMB_D + 64, :], uc_ref[...])
  h += dg(w1_ref[2 * EMB_D + 64:, :], ac_ref[...])
  h = jnp.maximum(h + b1_ref[...][:, None], 0.0)
  h = jnp.maximum(dg(w2_ref[...], h) + b2_ref[...][:, None], 0.0)
  h = jnp.maximum(dg(w3_ref[...], h) + b3_ref[...][:, None], 0.0)
  out = dg(w4_ref[...], h) + b4_ref[...][:, None]
  out_ref[...] = out[0, :]


def _mlp_call(embT, ucT, acT, W1, b1, W2, b2, W3, b3, W4, b4, blk):
  B = embT.shape[1]
  grid = (B // blk,)

  def col_spec(d):
    return pl.BlockSpec((d, blk), lambda i: (0, i))

  def full_spec(shape):
    nd = len(shape)
    return pl.BlockSpec(shape, lambda i: (0,) * nd)

  return pl.pallas_call(
      _mlp_body,
      grid=grid,
      in_specs=[
          col_spec(2 * EMB_D), col_spec(64), col_spec(64),
          full_spec(W1.shape), full_spec(b1.shape),
          full_spec(W2.shape), full_spec(b2.shape),
          full_spec(W3.shape), full_spec(b3.shape),
          full_spec(W4.shape), full_spec(b4.shape),
      ],
      out_specs=pl.BlockSpec((blk,), lambda i: (i,)),
      out_shape=jax.ShapeDtypeStruct((B,), jnp.float32),
  )(embT, ucT, acT, W1, b1, W2, b2, W3, b3, W4, b4)


@jax.jit
def kernel(user_id, artist_id, user_country, artist_country, U, T,
           W1, b1, W2, b2, W3, b3, W4, b4):
  B = user_id.shape[0]
  mesh = plsc.VectorSubcoreMesh(core_axis_name="c", subcore_axis_name="s")
  n_workers = mesh.num_cores * mesh.num_subcores
  gather = _make_sc_gather(B, n_workers, mesh.num_cores)
  embT = gather(U.T, user_id, T.T, artist_id).reshape(2 * EMB_D, B)
  return _mlp_call(embT, user_country.T, artist_country.T,
                   W1, b1, W2, b2, W3, b3, W4, b4, blk=2048)


# speedup vs baseline: 14.8474x; 1.1771x over previous
import functools

import jax
import jax.numpy as jnp
from jax import lax
from jax.experimental import pallas as pl
from jax.experimental.pallas import tpu as pltpu
from jax.experimental.pallas import tpu_sc as plsc

EMB_D = 8


def _make_sc_gather(B, n_workers, num_cores):
  b_per_w = B // n_workers
  mesh = plsc.VectorSubcoreMesh(core_axis_name="c", subcore_axis_name="s")

  @functools.partial(
      pl.kernel,
      mesh=mesh,
      compiler_params=pltpu.CompilerParams(use_tc_tiling_on_sc=True,
                                           needs_layout_passes=False,
                                           disable_semaphore_checks=True),
      out_type=jax.ShapeDtypeStruct((2 * EMB_D * B,), jnp.float32),
      scratch_types=[
          pltpu.VMEM((b_per_w,), jnp.int32),
          pltpu.VMEM((b_per_w,), jnp.int32),
          pltpu.VMEM((2, 16, EMB_D, 128), jnp.float32),
          pltpu.VMEM((2, 16, EMB_D, 128), jnp.float32),
          pltpu.VMEM((2 * EMB_D * b_per_w,), jnp.float32),
          pltpu.SemaphoreType.DMA,
          pltpu.SemaphoreType.DMA,
      ],
  )
  def sc_gather(ut_hbm, uid_hbm, tt_hbm, tid_hbm, out, uidx_v, tidx_v,
                tiles_u, tiles_t, cols_v, sem_u, sem_t):
    wid = lax.axis_index("s") * num_cores + lax.axis_index("c")
    base = wid * b_per_w
    pltpu.sync_copy(uid_hbm.at[pl.ds(base, b_per_w)], uidx_v)
    pltpu.sync_copy(tid_hbm.at[pl.ds(base, b_per_w)], tidx_v)

    lanej = lax.iota(jnp.int32, 16)
    n_chunks = b_per_w // 16

    def fire(t, slot):
      e0 = t * 16
      uv = uidx_v[pl.ds(e0, 16)]
      tv = tidx_v[pl.ds(e0, 16)]
      for j in range(16):
        tu = lax.shift_right_logical(uv[j], 7) * 128
        tt = lax.shift_right_logical(tv[j], 7) * 128
        pltpu.async_copy(ut_hbm.at[:, pl.ds(tu, 128)],
                         tiles_u.at[slot, j], sem_u)
        pltpu.async_copy(tt_hbm.at[:, pl.ds(tt, 128)],
                         tiles_t.at[slot, j], sem_t)

    def drain():
      for j in range(16):
        pltpu.make_async_copy(ut_hbm.at[:, pl.ds(0, 128)],
                              tiles_u.at[0, 0], sem_u).wait()
        pltpu.make_async_copy(tt_hbm.at[:, pl.ds(0, 128)],
                              tiles_t.at[0, 0], sem_t).wait()

    def extract(t, slot):
      e0 = t * 16
      uv = uidx_v[pl.ds(e0, 16)]
      tv = tidx_v[pl.ds(e0, 16)]
      cu = lax.bitwise_and(uv, 127)
      ct = lax.bitwise_and(tv, 127)
      for k in range(EMB_D):
        ksplat = jnp.full((16,), k, jnp.int32)
        valu = plsc.load_gather(tiles_u.at[slot], [lanej, ksplat, cu])
        valt = plsc.load_gather(tiles_t.at[slot], [lanej, ksplat, ct])
        cols_v[pl.ds(k * b_per_w + e0, 16)] = valu
        cols_v[pl.ds((EMB_D + k) * b_per_w + e0, 16)] = valt

    fire(0, 0)

    def body(p, _):
      t0 = p * 2
      fire(t0 + 1, 1)
      drain()
      extract(t0, 0)

      @pl.when(t0 + 2 < n_chunks)
      def _next():
        fire(t0 + 2, 0)

      drain()
      extract(t0 + 1, 1)
      return _

    lax.fori_loop(0, n_chunks // 2, body, None)

    for k in range(2 * EMB_D):
      pltpu.sync_copy(cols_v.at[pl.ds(k * b_per_w, b_per_w)],
                      out.at[pl.ds(k * B + base, b_per_w)])

  return sc_gather


def _mlp_body(emb_ref, uc_ref, ac_ref,
              w1_ref, b1_ref, w2_ref, b2_ref, w3_ref, b3_ref, w4_ref, b4_ref,
              out_ref):
  def dg(w, x):
    return lax.dot_general(w, x, (((0,), (0,)), ((), ())),
                           precision=lax.Precision.HIGHEST,
                           preferred_element_type=jnp.float32)

  h = dg(w1_ref[0:EMB_D, :], emb_ref[0:EMB_D, :])
  h += dg(w1_ref[EMB_D:2 * EMB_D, :], emb_ref[EMB_D:2 * EMB_D, :])
  h += dg(w1_ref[2 * EMB_D:2 * EMB_D + 64, :], uc_ref[...])
  h += dg(w1_ref[2 * EMB_D + 64:, :], ac_ref[...])
  h = jnp.maximum(h + b1_ref[...][:, None], 0.0)
  h = jnp.maximum(dg(w2_ref[...], h) + b2_ref[...][:, None], 0.0)
  h = jnp.maximum(dg(w3_ref[...], h) + b3_ref[...][:, None], 0.0)
  out = dg(w4_ref[...], h) + b4_ref[...][:, None]
  out_ref[...] = out[0, :]


def _mlp_call(embT, ucT, acT, W1, b1, W2, b2, W3, b3, W4, b4, blk):
  B = embT.shape[1]
  grid = (B // blk,)

  def col_spec(d):
    return pl.BlockSpec((d, blk), lambda i: (0, i))

  def full_spec(shape):
    nd = len(shape)
    return pl.BlockSpec(shape, lambda i: (0,) * nd)

  return pl.pallas_call(
      _mlp_body,
      grid=grid,
      in_specs=[
          col_spec(2 * EMB_D), col_spec(64), col_spec(64),
          full_spec(W1.shape), full_spec(b1.shape),
          full_spec(W2.shape), full_spec(b2.shape),
          full_spec(W3.shape), full_spec(b3.shape),
          full_spec(W4.shape), full_spec(b4.shape),
      ],
      out_specs=pl.BlockSpec((blk,), lambda i: (i,)),
      out_shape=jax.ShapeDtypeStruct((B,), jnp.float32),
  )(embT, ucT, acT, W1, b1, W2, b2, W3, b3, W4, b4)


@jax.jit
def kernel(user_id, artist_id, user_country, artist_country, U, T,
           W1, b1, W2, b2, W3, b3, W4, b4):
  B = user_id.shape[0]
  mesh = plsc.VectorSubcoreMesh(core_axis_name="c", subcore_axis_name="s")
  n_workers = mesh.num_cores * mesh.num_subcores
  gather = _make_sc_gather(B, n_workers, mesh.num_cores)
  embT = gather(U.T, user_id, T.T, artist_id).reshape(2 * EMB_D, B)
  return _mlp_call(embT, user_country.T, artist_country.T,
                   W1, b1, W2, b2, W3, b3, W4, b4, blk=2048)


# speedup vs baseline: 16.4859x; 1.1104x over previous
import functools

import jax
import jax.numpy as jnp
from jax import lax
from jax.experimental import pallas as pl
from jax.experimental.pallas import tpu as pltpu
from jax.experimental.pallas import tpu_sc as plsc

EMB_D = 8


def _make_sc_gather(B, n_workers, num_cores):
  b_per_w = B // n_workers
  mesh = plsc.VectorSubcoreMesh(core_axis_name="c", subcore_axis_name="s")

  @functools.partial(
      pl.kernel,
      mesh=mesh,
      compiler_params=pltpu.CompilerParams(use_tc_tiling_on_sc=True,
                                           needs_layout_passes=False,
                                           disable_semaphore_checks=True),
      out_type=jax.ShapeDtypeStruct((2 * EMB_D * B,), jnp.float32),
      scratch_types=[
          pltpu.VMEM((b_per_w,), jnp.int32),
          pltpu.VMEM((b_per_w,), jnp.int32),
          pltpu.VMEM((2, 16, EMB_D, 128), jnp.float32),
          pltpu.VMEM((2, 16, EMB_D, 128), jnp.float32),
          pltpu.VMEM((2 * EMB_D * b_per_w,), jnp.float32),
          pltpu.SemaphoreType.DMA,
          pltpu.SemaphoreType.DMA,
      ],
  )
  def sc_gather(ut_hbm, uid_hbm, tt_hbm, tid_hbm, out, uidx_v, tidx_v,
                tiles_u, tiles_t, cols_v, sem_u, sem_t):
    wid = lax.axis_index("s") * num_cores + lax.axis_index("c")
    base = wid * b_per_w
    pltpu.sync_copy(uid_hbm.at[pl.ds(base, b_per_w)], uidx_v)
    pltpu.sync_copy(tid_hbm.at[pl.ds(base, b_per_w)], tidx_v)

    lanej = lax.iota(jnp.int32, 16)
    n_chunks = b_per_w // 16

    def fire(t, slot):
      e0 = t * 16
      uv = uidx_v[pl.ds(e0, 16)]
      tv = tidx_v[pl.ds(e0, 16)]
      for j in range(16):
        tu = lax.shift_right_logical(uv[j], 7) * 128
        tt = lax.shift_right_logical(tv[j], 7) * 128
        pltpu.async_copy(ut_hbm.at[:, pl.ds(tu, 128)],
                         tiles_u.at[slot, j], sem_u)
        pltpu.async_copy(tt_hbm.at[:, pl.ds(tt, 128)],
                         tiles_t.at[slot, j], sem_t)

    def drain():
      for j in range(16):
        pltpu.make_async_copy(ut_hbm.at[:, pl.ds(0, 128)],
                              tiles_u.at[0, 0], sem_u).wait()
        pltpu.make_async_copy(tt_hbm.at[:, pl.ds(0, 128)],
                              tiles_t.at[0, 0], sem_t).wait()

    def extract(t, slot):
      e0 = t * 16
      uv = uidx_v[pl.ds(e0, 16)]
      tv = tidx_v[pl.ds(e0, 16)]
      cu = lax.bitwise_and(uv, 127)
      ct = lax.bitwise_and(tv, 127)
      for k in range(EMB_D):
        ksplat = jnp.full((16,), k, jnp.int32)
        valu = plsc.load_gather(tiles_u.at[slot], [lanej, ksplat, cu])
        valt = plsc.load_gather(tiles_t.at[slot], [lanej, ksplat, ct])
        cols_v[pl.ds(k * b_per_w + e0, 16)] = valu
        cols_v[pl.ds((EMB_D + k) * b_per_w + e0, 16)] = valt

    fire(0, 0)

    def body(p, _):
      t0 = p * 2
      fire(t0 + 1, 1)
      drain()
      extract(t0, 0)

      @pl.when(t0 + 2 < n_chunks)
      def _next():
        fire(t0 + 2, 0)

      drain()
      extract(t0 + 1, 1)
      return _

    lax.fori_loop(0, n_chunks // 2, body, None)

    for k in range(2 * EMB_D):
      pltpu.sync_copy(cols_v.at[pl.ds(k * b_per_w, b_per_w)],
                      out.at[pl.ds(k * B + base, b_per_w)])

  return sc_gather


def _mlp_body(emb_ref, uc_ref, ac_ref,
              w1_ref, b1_ref, w2_ref, b2_ref, w3_ref, b3_ref, w4_ref, b4_ref,
              out_ref):
  def dg(w, x):
    return lax.dot_general(w, x, (((0,), (0,)), ((), ())),
                           preferred_element_type=jnp.float32)

  h = dg(w1_ref[0:EMB_D, :], emb_ref[0:EMB_D, :])
  h += dg(w1_ref[EMB_D:2 * EMB_D, :], emb_ref[EMB_D:2 * EMB_D, :])
  h += dg(w1_ref[2 * EMB_D:2 * EMB_D + 64, :], uc_ref[...])
  h += dg(w1_ref[2 * EMB_D + 64:, :], ac_ref[...])
  h = jnp.maximum(h + b1_ref[...][:, None], 0.0)
  h = jnp.maximum(dg(w2_ref[...], h) + b2_ref[...][:, None], 0.0)
  h = jnp.maximum(dg(w3_ref[...], h) + b3_ref[...][:, None], 0.0)
  out = dg(w4_ref[...], h) + b4_ref[...][:, None]
  out_ref[...] = out[0, :]


def _mlp_call(embT, ucT, acT, W1, b1, W2, b2, W3, b3, W4, b4, blk):
  B = embT.shape[1]
  grid = (B // blk,)

  def col_spec(d):
    return pl.BlockSpec((d, blk), lambda i: (0, i))

  def full_spec(shape):
    nd = len(shape)
    return pl.BlockSpec(shape, lambda i: (0,) * nd)

  return pl.pallas_call(
      _mlp_body,
      grid=grid,
      in_specs=[
          col_spec(2 * EMB_D), col_spec(64), col_spec(64),
          full_spec(W1.shape), full_spec(b1.shape),
          full_spec(W2.shape), full_spec(b2.shape),
          full_spec(W3.shape), full_spec(b3.shape),
          full_spec(W4.shape), full_spec(b4.shape),
      ],
      out_specs=pl.BlockSpec((blk,), lambda i: (i,)),
      out_shape=jax.ShapeDtypeStruct((B,), jnp.float32),
  )(embT, ucT, acT, W1, b1, W2, b2, W3, b3, W4, b4)


@jax.jit
def kernel(user_id, artist_id, user_country, artist_country, U, T,
           W1, b1, W2, b2, W3, b3, W4, b4):
  B = user_id.shape[0]
  mesh = plsc.VectorSubcoreMesh(core_axis_name="c", subcore_axis_name="s")
  n_workers = mesh.num_cores * mesh.num_subcores
  gather = _make_sc_gather(B, n_workers, mesh.num_cores)
  embT = gather(U.T, user_id, T.T, artist_id).reshape(2 * EMB_D, B)
  return _mlp_call(embT, user_country.T, artist_country.T,
                   W1, b1, W2, b2, W3, b3, W4, b4, blk=4096)


# speedup vs baseline: 16.6232x; 1.0083x over previous
import functools

import jax
import jax.numpy as jnp
from jax import lax
from jax.experimental import pallas as pl
from jax.experimental.pallas import tpu as pltpu
from jax.experimental.pallas import tpu_sc as plsc

EMB_D = 8
HIDDEN0 = 16


def _make_sc_gather(B, n_workers, num_cores):
  b_per_w = B // n_workers
  mesh = plsc.VectorSubcoreMesh(core_axis_name="c", subcore_axis_name="s")

  @functools.partial(
      pl.kernel,
      mesh=mesh,
      compiler_params=pltpu.CompilerParams(use_tc_tiling_on_sc=True,
                                           needs_layout_passes=False,
                                           disable_semaphore_checks=True),
      out_type=jax.ShapeDtypeStruct((2 * EMB_D * B,), jnp.float32),
      scratch_types=[
          pltpu.VMEM((b_per_w,), jnp.int32),
          pltpu.VMEM((b_per_w,), jnp.int32),
          pltpu.VMEM((2, 16, EMB_D, 128), jnp.float32),
          pltpu.VMEM((2, 16, EMB_D, 128), jnp.float32),
          pltpu.VMEM((2 * EMB_D * b_per_w,), jnp.float32),
          pltpu.SemaphoreType.DMA,
          pltpu.SemaphoreType.DMA,
      ],
  )
  def sc_gather(ut_hbm, uid_hbm, tt_hbm, tid_hbm, out, uidx_v, tidx_v,
                tiles_u, tiles_t, cols_v, sem_u, sem_t):
    wid = lax.axis_index("s") * num_cores + lax.axis_index("c")
    base = wid * b_per_w
    pltpu.sync_copy(uid_hbm.at[pl.ds(base, b_per_w)], uidx_v)
    pltpu.sync_copy(tid_hbm.at[pl.ds(base, b_per_w)], tidx_v)

    lanej = lax.iota(jnp.int32, 16)
    n_chunks = b_per_w // 16

    def fire(t, slot):
      e0 = t * 16
      uv = uidx_v[pl.ds(e0, 16)]
      tv = tidx_v[pl.ds(e0, 16)]
      for j in range(16):
        tu = lax.shift_right_logical(uv[j], 7) * 128
        tt = lax.shift_right_logical(tv[j], 7) * 128
        pltpu.async_copy(ut_hbm.at[:, pl.ds(tu, 128)],
                         tiles_u.at[slot, j], sem_u)
        pltpu.async_copy(tt_hbm.at[:, pl.ds(tt, 128)],
                         tiles_t.at[slot, j], sem_t)

    def drain():
      for j in range(16):
        pltpu.make_async_copy(ut_hbm.at[:, pl.ds(0, 128)],
                              tiles_u.at[0, 0], sem_u).wait()
        pltpu.make_async_copy(tt_hbm.at[:, pl.ds(0, 128)],
                              tiles_t.at[0, 0], sem_t).wait()

    def extract(t, slot):
      e0 = t * 16
      uv = uidx_v[pl.ds(e0, 16)]
      tv = tidx_v[pl.ds(e0, 16)]
      cu = lax.bitwise_and(uv, 127)
      ct = lax.bitwise_and(tv, 127)
      for k in range(EMB_D):
        ksplat = jnp.full((16,), k, jnp.int32)
        valu = plsc.load_gather(tiles_u.at[slot], [lanej, ksplat, cu])
        valt = plsc.load_gather(tiles_t.at[slot], [lanej, ksplat, ct])
        cols_v[pl.ds(k * b_per_w + e0, 16)] = valu
        cols_v[pl.ds((EMB_D + k) * b_per_w + e0, 16)] = valt

    fire(0, 0)

    def body(p, _):
      t0 = p * 2
      fire(t0 + 1, 1)
      drain()
      extract(t0, 0)

      @pl.when(t0 + 2 < n_chunks)
      def _next():
        fire(t0 + 2, 0)

      drain()
      extract(t0 + 1, 1)
      return _

    lax.fori_loop(0, n_chunks // 2, body, None)

    for k in range(2 * EMB_D):
      pltpu.sync_copy(cols_v.at[pl.ds(k * b_per_w, b_per_w)],
                      out.at[pl.ds(k * B + base, b_per_w)])

  return sc_gather


def _dg(w, x):
  return lax.dot_general(w, x, (((0,), (0,)), ((), ())),
                         preferred_element_type=jnp.float32)


def _country_body(uc_ref, ac_ref, w1_ref, b1_ref, out_ref):
  h = _dg(w1_ref[2 * EMB_D:2 * EMB_D + 64, :], uc_ref[...])
  h += _dg(w1_ref[2 * EMB_D + 64:, :], ac_ref[...])
  out_ref[...] = h + b1_ref[...][:, None]


def _head_body(emb_ref, cpart_ref, w1_ref, w2_ref, b2_ref, w3_ref, b3_ref,
               w4_ref, b4_ref, out_ref):
  h = _dg(w1_ref[0:EMB_D, :], emb_ref[0:EMB_D, :])
  h += _dg(w1_ref[EMB_D:2 * EMB_D, :], emb_ref[EMB_D:2 * EMB_D, :])
  h = jnp.maximum(h + cpart_ref[...], 0.0)
  h = jnp.maximum(_dg(w2_ref[...], h) + b2_ref[...][:, None], 0.0)
  h = jnp.maximum(_dg(w3_ref[...], h) + b3_ref[...][:, None], 0.0)
  out = _dg(w4_ref[...], h) + b4_ref[...][:, None]
  out_ref[...] = out[0, :]


def _col_spec(d, blk):
  return pl.BlockSpec((d, blk), lambda i: (0, i))


def _full_spec(shape):
  nd = len(shape)
  return pl.BlockSpec(shape, lambda i: (0,) * nd)


def _country_call(ucT, acT, W1, b1, blk):
  B = ucT.shape[1]
  return pl.pallas_call(
      _country_body,
      grid=(B // blk,),
      in_specs=[
          _col_spec(64, blk), _col_spec(64, blk),
          _full_spec(W1.shape), _full_spec(b1.shape),
      ],
      out_specs=_col_spec(HIDDEN0, blk),
      out_shape=jax.ShapeDtypeStruct((HIDDEN0, B), jnp.float32),
  )(ucT, acT, W1, b1)


def _head_call(embT, cpart, W1, W2, b2, W3, b3, W4, b4, blk):
  B = embT.shape[1]
  return pl.pallas_call(
      _head_body,
      grid=(B // blk,),
      in_specs=[
          _col_spec(2 * EMB_D, blk), _col_spec(HIDDEN0, blk),
          _full_spec(W1.shape), _full_spec(W2.shape), _full_spec(b2.shape),
          _full_spec(W3.shape), _full_spec(b3.shape),
          _full_spec(W4.shape), _full_spec(b4.shape),
      ],
      out_specs=pl.BlockSpec((blk,), lambda i: (i,)),
      out_shape=jax.ShapeDtypeStruct((B,), jnp.float32),
  )(embT, cpart, W1, W2, b2, W3, b3, W4, b4)


@jax.jit
def kernel(user_id, artist_id, user_country, artist_country, U, T,
           W1, b1, W2, b2, W3, b3, W4, b4):
  B = user_id.shape[0]
  mesh = plsc.VectorSubcoreMesh(core_axis_name="c", subcore_axis_name="s")
  n_workers = mesh.num_cores * mesh.num_subcores
  gather = _make_sc_gather(B, n_workers, mesh.num_cores)
  embT = gather(U.T, user_id, T.T, artist_id).reshape(2 * EMB_D, B)
  cpart = _country_call(user_country.T, artist_country.T, W1, b1, blk=4096)
  return _head_call(embT, cpart, W1, W2, b2, W3, b3, W4, b4, blk=4096)
